# baseline (speedup 1.0000x reference)
import functools

import jax
import jax.numpy as jnp
from jax import lax
from jax.experimental import pallas as pl
from jax.experimental.pallas import tpu as pltpu
from jax.experimental.pallas import tpu_sc as plsc

_TOPK = 25
_DROP = 3
_KEEP = _TOPK - _DROP

_B = 64
_C = 1000
_S = 24
_S_SC = 22
_S_TC = _S - _S_SC

_ROWS = _B * _S * _S

_GRID = 16
_BB = _B // _GRID
_PR = _ROWS // _GRID

_NW = 32
_L = 16
_NCH = 63

_NBUF = 4
_NCHUNK = _S_SC
_NITER = 2 * _NCHUNK


def _fused_body(mo_ref, p_ref, f_ref, sum_ref, gsum_ref, pacc_ref, gacc_ref,
                mask_ref):
    i = pl.program_id(0)

    @pl.when(i == 0)
    def _():
        pacc_ref[...] = jnp.zeros_like(pacc_ref)
        gacc_ref[...] = jnp.zeros_like(gacc_ref)
        vals = mo_ref[...]
        col = lax.broadcasted_iota(jnp.int32, (_B, _C), 1)
        mask = jnp.zeros((_B, _C), jnp.float32)
        for t in range(_TOPK):
            m = jnp.max(vals, axis=1, keepdims=True)
            cand = jnp.where(vals == m, col, jnp.int32(_C))
            amin = jnp.min(cand, axis=1, keepdims=True)
            hit = col == amin
            if t >= _DROP:
                mask = mask + jnp.where(hit, jnp.float32(1.0), jnp.float32(0.0))
            vals = jnp.where(hit, -jnp.inf, vals)
        mask_ref[...] = mask.reshape(_GRID, _BB, _C)

    x = p_ref[...].reshape(_PR // 8, 8, _C)
    pacc_ref[...] += jnp.sum(x, axis=0)

    f = f_ref[...].reshape(_BB, _S_TC * _S, _C)
    m = mask_ref[i].reshape(_BB, 1, _C)
    fm = (f * m).reshape(_BB * _S_TC * _S // 8, 8, _C)
    gacc_ref[...] += jnp.sum(fm, axis=0)

    @pl.when(i == _GRID - 1)
    def _():
        sum_ref[0, 0] = jnp.sum(pacc_ref[...])
        gsum_ref[0, 0] = jnp.sum(gacc_ref[...])


def _psum_feat(main_out, pt, ft):
    return pl.pallas_call(
        _fused_body,
        grid=(_GRID,),
        in_specs=[
            pl.BlockSpec((_B, _C), lambda i: (0, 0)),
            pl.BlockSpec((_PR, _C), lambda i: (i, 0)),
            pl.BlockSpec((_BB, _S_TC, _S, _C), lambda i: (i, _S_SC // _S_TC, 0, 0)),
        ],
        out_specs=[
            pl.BlockSpec(memory_space=pltpu.SMEM),
            pl.BlockSpec(memory_space=pltpu.SMEM),
        ],
        out_shape=[
            jax.ShapeDtypeStruct((1, 1), jnp.float32),
            jax.ShapeDtypeStruct((1, 1), jnp.float32),
        ],
        scratch_shapes=[
            pltpu.VMEM((8, _C), jnp.float32),
            pltpu.VMEM((8, _C), jnp.float32),
            pltpu.VMEM((_GRID, _BB, _C), jnp.float32),
        ],
    )(main_out, pt, ft)


def _sc_gather_sum(ft, main_out):
    mesh = plsc.VectorSubcoreMesh(core_axis_name="c", subcore_axis_name="s")

    @functools.partial(
        pl.kernel,
        mesh=mesh,
        compiler_params=pltpu.CompilerParams(needs_layout_passes=False),
        out_type=jax.ShapeDtypeStruct((_NW, _L), jnp.float32),
        scratch_types=(
            [pltpu.VMEM((8, _C), jnp.float32)]
            + [pltpu.VMEM((1, _S, _C), jnp.float32)] * _NBUF
            + [pltpu.VMEM((_L,), jnp.float32)]
            + [pltpu.SemaphoreType.DMA] * _NBUF
        ),
    )
    def k(ft_hbm, mo_hbm, out_hbm, mov, *rest):
        bufs = rest[:_NBUF]
        accv = rest[_NBUF]
        sems = rest[_NBUF + 1 :]
        w = lax.axis_index("s") * 2 + lax.axis_index("c")
        lane = lax.iota(jnp.int32, _L)
        neg_inf = jnp.full((_L,), -jnp.inf, jnp.float32)

        base = (2 * w) // 8 * 8
        pltpu.sync_copy(mo_hbm.at[pl.ds(base, 8)], mov)

        def start(t, slot):
            b_i, chunk = divmod(t, _NCHUNK)
            return pltpu.async_copy(
                ft_hbm.at[2 * w + b_i, pl.ds(chunk, 1)], bufs[slot], sems[slot]
            )

        cps = [start(t, t) for t in range(_NBUF)]

        def topk_row(r):
            rv = jnp.full((_L,), r, jnp.int32)

            def lane_chunk(g, l0):
                cv = l0 + _L * (_L * g + lane)
                v = plsc.load_gather(mov, [rv, jnp.minimum(cv, _C - 1)])
                return jnp.where(cv < _C, v, neg_inf), cv

            vmax = neg_inf
            vidx = jnp.zeros((_L,), jnp.int32)
            for c in range(_NCH):
                cv = _L * c + lane
                v = plsc.load_gather(mov, [rv, jnp.minimum(cv, _C - 1)])
                v = jnp.where(cv < _C, v, neg_inf)
                upd = v > vmax
                vmax = jnp.where(upd, v, vmax)
                vidx = jnp.where(upd, cv, vidx)

            def step(t, carry):
                vmax, vidx, ca, cb = carry
                m = lax.reduce_max(vmax, (0,))
                cstar = lax.reduce_min(
                    jnp.where(vmax == m, vidx, jnp.int32(2 * _C)), (0,))
                j = t - _DROP
                ca = jnp.where((j >= 0) & (lane == j), cstar, ca)
                cb = jnp.where(lane == j - _L, cstar, cb)
                lstar = cstar % _L
                plsc.store_scatter(mov, [rv, jnp.full((_L,), cstar, jnp.int32)],
                                   neg_inf, mask=lane == 0)
                lm = neg_inf
                li = jnp.zeros((_L,), jnp.int32)
                for g in range(4):
                    v, cv = lane_chunk(g, lstar)
                    upd = v > lm
                    lm = jnp.where(upd, v, lm)
                    li = jnp.where(upd, cv, li)
                ml = lax.reduce_max(lm, (0,))
                cl = lax.reduce_min(
                    jnp.where(lm == ml, li, jnp.int32(2 * _C)), (0,))
                vmax = jnp.where(lane == lstar, ml, vmax)
                vidx = jnp.where(lane == lstar, cl, vidx)
                return vmax, vidx, ca, cb

            zi = jnp.zeros((_L,), jnp.int32)
            _, _, ca, cb = lax.fori_loop(
                0, _TOPK, step, (vmax, vidx, zi, zi))
            return ca, cb

        r0 = 2 * w - base
        ca0, cb0 = topk_row(r0)
        ca1, cb1 = topk_row(r0 + 1)

        tail_on = lane < (_KEEP - _L)
        z = jnp.zeros((_L,), jnp.int32)

        def reduce_chunk(t, slot, total):
            b_i = t // _NCHUNK
            buf = bufs[slot]
            ca = jnp.where(b_i == 0, ca0, ca1)
            cb = jnp.where(b_i == 0, cb0, cb1)
            cbc = jnp.minimum(cb, _C - 1)

            def srow(s2, acc):
                s2v = jnp.broadcast_to(s2, (_L,)).astype(jnp.int32)
                ga = plsc.load_gather(buf, [z, s2v, ca])
                gb = plsc.load_gather(buf, [z, s2v, cbc])
                return acc + ga + jnp.where(tail_on, gb, jnp.float32(0.0))

            return lax.fori_loop(0, _S, srow, total)

        total = jnp.zeros((_L,), jnp.float32)
        for t in range(_NITER):
            cps[t % _NBUF].wait()
            if t + _NBUF < _NITER:
                cps[t % _NBUF] = start(t + _NBUF, t % _NBUF)
            total = reduce_chunk(t, t % _NBUF, total)

        accv[...] = total
        pltpu.sync_copy(accv, out_hbm.at[w])

    return k(ft, main_out)


def kernel(p, main_out, features):
    pt = jnp.transpose(p, (0, 2, 3, 1)).reshape(_ROWS, _C)
    ft = jnp.transpose(features, (0, 2, 3, 1))
    partials = _sc_gather_sum(ft, main_out)
    sum_p, gsum_tc = _psum_feat(main_out, pt, ft)
    denom = jnp.float32(_B * _S * _S)
    gsum = gsum_tc[0, 0] + jnp.sum(partials)
    loss = sum_p[0, 0] / denom + gsum / (denom * _KEEP)
    return jnp.abs(loss)


# speedup vs baseline: 1.0156x; 1.0156x over previous
import functools

import jax
import jax.numpy as jnp
from jax import lax
from jax.experimental import pallas as pl
from jax.experimental.pallas import tpu as pltpu
from jax.experimental.pallas import tpu_sc as plsc

_TOPK = 25
_DROP = 3
_KEEP = _TOPK - _DROP

_B = 64
_C = 1000
_S = 24
_S_SC = 21
_S_TC = _S - _S_SC

_ROWS = _B * _S * _S

_GRID = 8
_BB = _B // _GRID
_PR = _ROWS // _GRID

_NW = 32
_L = 16
_NCH = 63

_NBUF = 4
_NCHUNK = _S_SC
_NITER = 2 * _NCHUNK


def _fused_body(mo_ref, p_ref, f_ref, sum_ref, gsum_ref, pacc_ref, gacc_ref,
                mask_ref):
    i = pl.program_id(0)

    @pl.when(i == 0)
    def _():
        pacc_ref[...] = jnp.zeros_like(pacc_ref)
        gacc_ref[...] = jnp.zeros_like(gacc_ref)
        vals = mo_ref[...]
        col = lax.broadcasted_iota(jnp.int32, (_B, _C), 1)
        mask = jnp.zeros((_B, _C), jnp.float32)
        for t in range(_TOPK):
            m = jnp.max(vals, axis=1, keepdims=True)
            cand = jnp.where(vals == m, col, jnp.int32(_C))
            amin = jnp.min(cand, axis=1, keepdims=True)
            hit = col == amin
            if t >= _DROP:
                mask = mask + jnp.where(hit, jnp.float32(1.0), jnp.float32(0.0))
            vals = jnp.where(hit, -jnp.inf, vals)
        mask_ref[...] = mask.reshape(_GRID, _BB, _C)

    x = p_ref[...].reshape(_PR // 8, 8, _C)
    pacc_ref[...] += jnp.sum(x, axis=0)

    f = f_ref[...].reshape(_BB, _S_TC * _S, _C)
    m = mask_ref[i].reshape(_BB, 1, _C)
    fm = (f * m).reshape(_BB * _S_TC * _S // 8, 8, _C)
    gacc_ref[...] += jnp.sum(fm, axis=0)

    @pl.when(i == _GRID - 1)
    def _():
        sum_ref[0, 0] = jnp.sum(pacc_ref[...])
        gsum_ref[0, 0] = jnp.sum(gacc_ref[...])


def _psum_feat(main_out, pt, ft):
    return pl.pallas_call(
        _fused_body,
        grid=(_GRID,),
        in_specs=[
            pl.BlockSpec((_B, _C), lambda i: (0, 0)),
            pl.BlockSpec((_PR, _C), lambda i: (i, 0)),
            pl.BlockSpec((_BB, _S_TC, _S, _C), lambda i: (i, _S_SC // _S_TC, 0, 0)),
        ],
        out_specs=[
            pl.BlockSpec(memory_space=pltpu.SMEM),
            pl.BlockSpec(memory_space=pltpu.SMEM),
        ],
        out_shape=[
            jax.ShapeDtypeStruct((1, 1), jnp.float32),
            jax.ShapeDtypeStruct((1, 1), jnp.float32),
        ],
        scratch_shapes=[
            pltpu.VMEM((8, _C), jnp.float32),
            pltpu.VMEM((8, _C), jnp.float32),
            pltpu.VMEM((_GRID, _BB, _C), jnp.float32),
        ],
    )(main_out, pt, ft)


def _sc_gather_sum(ft, main_out):
    mesh = plsc.VectorSubcoreMesh(core_axis_name="c", subcore_axis_name="s")

    @functools.partial(
        pl.kernel,
        mesh=mesh,
        compiler_params=pltpu.CompilerParams(needs_layout_passes=False),
        out_type=jax.ShapeDtypeStruct((_NW, _L), jnp.float32),
        scratch_types=(
            [pltpu.VMEM((8, _C), jnp.float32)]
            + [pltpu.VMEM((1, _S, _C), jnp.float32)] * _NBUF
            + [pltpu.VMEM((_L,), jnp.float32)]
            + [pltpu.SemaphoreType.DMA] * _NBUF
        ),
    )
    def k(ft_hbm, mo_hbm, out_hbm, mov, *rest):
        bufs = rest[:_NBUF]
        accv = rest[_NBUF]
        sems = rest[_NBUF + 1 :]
        w = lax.axis_index("s") * 2 + lax.axis_index("c")
        lane = lax.iota(jnp.int32, _L)
        neg_inf = jnp.full((_L,), -jnp.inf, jnp.float32)

        base = (2 * w) // 8 * 8
        pltpu.sync_copy(mo_hbm.at[pl.ds(base, 8)], mov)

        def start(t, slot):
            b_i, chunk = divmod(t, _NCHUNK)
            return pltpu.async_copy(
                ft_hbm.at[2 * w + b_i, pl.ds(chunk, 1)], bufs[slot], sems[slot]
            )

        cps = [start(t, t) for t in range(_NBUF)]

        def topk_row(r):
            rv = jnp.full((_L,), r, jnp.int32)

            def lane_chunk(g, l0):
                cv = l0 + _L * (_L * g + lane)
                v = plsc.load_gather(mov, [rv, jnp.minimum(cv, _C - 1)])
                return jnp.where(cv < _C, v, neg_inf), cv

            vmax = neg_inf
            vidx = jnp.zeros((_L,), jnp.int32)
            for c in range(_NCH):
                cv = _L * c + lane
                v = plsc.load_gather(mov, [rv, jnp.minimum(cv, _C - 1)])
                v = jnp.where(cv < _C, v, neg_inf)
                upd = v > vmax
                vmax = jnp.where(upd, v, vmax)
                vidx = jnp.where(upd, cv, vidx)

            def step(t, carry):
                vmax, vidx, ca, cb = carry
                m = lax.reduce_max(vmax, (0,))
                cstar = lax.reduce_min(
                    jnp.where(vmax == m, vidx, jnp.int32(2 * _C)), (0,))
                j = t - _DROP
                ca = jnp.where((j >= 0) & (lane == j), cstar, ca)
                cb = jnp.where(lane == j - _L, cstar, cb)
                lstar = cstar % _L
                plsc.store_scatter(mov, [rv, jnp.full((_L,), cstar, jnp.int32)],
                                   neg_inf, mask=lane == 0)
                lm = neg_inf
                li = jnp.zeros((_L,), jnp.int32)
                for g in range(4):
                    v, cv = lane_chunk(g, lstar)
                    upd = v > lm
                    lm = jnp.where(upd, v, lm)
                    li = jnp.where(upd, cv, li)
                ml = lax.reduce_max(lm, (0,))
                cl = lax.reduce_min(
                    jnp.where(lm == ml, li, jnp.int32(2 * _C)), (0,))
                vmax = jnp.where(lane == lstar, ml, vmax)
                vidx = jnp.where(lane == lstar, cl, vidx)
                return vmax, vidx, ca, cb

            zi = jnp.zeros((_L,), jnp.int32)
            _, _, ca, cb = lax.fori_loop(
                0, _TOPK, step, (vmax, vidx, zi, zi))
            return ca, cb

        r0 = 2 * w - base
        ca0, cb0 = topk_row(r0)
        ca1, cb1 = topk_row(r0 + 1)

        tail_on = lane < (_KEEP - _L)
        z = jnp.zeros((_L,), jnp.int32)

        def reduce_chunk(t, slot, total):
            b_i = t // _NCHUNK
            buf = bufs[slot]
            ca = jnp.where(b_i == 0, ca0, ca1)
            cb = jnp.where(b_i == 0, cb0, cb1)
            cbc = jnp.minimum(cb, _C - 1)

            def srow(s2, acc):
                s2v = jnp.broadcast_to(s2, (_L,)).astype(jnp.int32)
                ga = plsc.load_gather(buf, [z, s2v, ca])
                gb = plsc.load_gather(buf, [z, s2v, cbc])
                return acc + ga + jnp.where(tail_on, gb, jnp.float32(0.0))

            return lax.fori_loop(0, _S, srow, total)

        total = jnp.zeros((_L,), jnp.float32)
        for t in range(_NITER):
            cps[t % _NBUF].wait()
            if t + _NBUF < _NITER:
                cps[t % _NBUF] = start(t + _NBUF, t % _NBUF)
            total = reduce_chunk(t, t % _NBUF, total)

        accv[...] = total
        pltpu.sync_copy(accv, out_hbm.at[w])

    return k(ft, main_out)


def kernel(p, main_out, features):
    pt = jnp.transpose(p, (0, 2, 3, 1)).reshape(_ROWS, _C)
    ft = jnp.transpose(features, (0, 2, 3, 1))
    partials = _sc_gather_sum(ft, main_out)
    sum_p, gsum_tc = _psum_feat(main_out, pt, ft)
    denom = jnp.float32(_B * _S * _S)
    gsum = gsum_tc[0, 0] + jnp.sum(partials)
    loss = sum_p[0, 0] / denom + gsum / (denom * _KEEP)
    return jnp.abs(loss)


# speedup vs baseline: 1.0196x; 1.0039x over previous
import functools

import jax
import jax.numpy as jnp
from jax import lax
from jax.experimental import pallas as pl
from jax.experimental.pallas import tpu as pltpu
from jax.experimental.pallas import tpu_sc as plsc

_TOPK = 25
_DROP = 3
_KEEP = _TOPK - _DROP

_B = 64
_C = 1000
_S = 24
_S_SC = 20
_S_TC = _S - _S_SC

_ROWS = _B * _S * _S

_GRID = 8
_BB = _B // _GRID
_PR = _ROWS // _GRID

_NW = 32
_L = 16
_NCH = 63

_NBUF = 4
_NCHUNK = _S_SC
_NITER = 2 * _NCHUNK


def _fused_body(mo_ref, p_ref, f_ref, sum_ref, gsum_ref, pacc_ref, gacc_ref,
                mask_ref):
    i = pl.program_id(0)

    @pl.when(i == 0)
    def _():
        pacc_ref[...] = jnp.zeros_like(pacc_ref)
        gacc_ref[...] = jnp.zeros_like(gacc_ref)
        vals = mo_ref[...]
        col = lax.broadcasted_iota(jnp.int32, (_B, _C), 1)
        mask = jnp.zeros((_B, _C), jnp.float32)
        for t in range(_TOPK):
            m = jnp.max(vals, axis=1, keepdims=True)
            cand = jnp.where(vals == m, col, jnp.int32(_C))
            amin = jnp.min(cand, axis=1, keepdims=True)
            hit = col == amin
            if t >= _DROP:
                mask = mask + jnp.where(hit, jnp.float32(1.0), jnp.float32(0.0))
            vals = jnp.where(hit, -jnp.inf, vals)
        mask_ref[...] = mask.reshape(_GRID, _BB, _C)

    x = p_ref[...].reshape(_PR // 8, 8, _C)
    pacc_ref[...] += jnp.sum(x, axis=0)

    f = f_ref[...].reshape(_BB, _S_TC * _S, _C)
    m = mask_ref[i].reshape(_BB, 1, _C)
    fm = (f * m).reshape(_BB * _S_TC * _S // 8, 8, _C)
    gacc_ref[...] += jnp.sum(fm, axis=0)

    @pl.when(i == _GRID - 1)
    def _():
        sum_ref[0, 0] = jnp.sum(pacc_ref[...])
        gsum_ref[0, 0] = jnp.sum(gacc_ref[...])


def _psum_feat(main_out, pt, ft):
    return pl.pallas_call(
        _fused_body,
        grid=(_GRID,),
        in_specs=[
            pl.BlockSpec((_B, _C), lambda i: (0, 0)),
            pl.BlockSpec((_PR, _C), lambda i: (i, 0)),
            pl.BlockSpec((_BB, _S_TC, _S, _C), lambda i: (i, _S_SC // _S_TC, 0, 0)),
        ],
        out_specs=[
            pl.BlockSpec(memory_space=pltpu.SMEM),
            pl.BlockSpec(memory_space=pltpu.SMEM),
        ],
        out_shape=[
            jax.ShapeDtypeStruct((1, 1), jnp.float32),
            jax.ShapeDtypeStruct((1, 1), jnp.float32),
        ],
        scratch_shapes=[
            pltpu.VMEM((8, _C), jnp.float32),
            pltpu.VMEM((8, _C), jnp.float32),
            pltpu.VMEM((_GRID, _BB, _C), jnp.float32),
        ],
    )(main_out, pt, ft)


def _sc_gather_sum(ft, main_out):
    mesh = plsc.VectorSubcoreMesh(core_axis_name="c", subcore_axis_name="s")

    @functools.partial(
        pl.kernel,
        mesh=mesh,
        compiler_params=pltpu.CompilerParams(needs_layout_passes=False),
        out_type=jax.ShapeDtypeStruct((_NW, _L), jnp.float32),
        scratch_types=(
            [pltpu.VMEM((8, _C), jnp.float32)]
            + [pltpu.VMEM((1, _S, _C), jnp.float32)] * _NBUF
            + [pltpu.VMEM((_L,), jnp.float32)]
            + [pltpu.SemaphoreType.DMA] * _NBUF
        ),
    )
    def k(ft_hbm, mo_hbm, out_hbm, mov, *rest):
        bufs = rest[:_NBUF]
        accv = rest[_NBUF]
        sems = rest[_NBUF + 1 :]
        w = lax.axis_index("s") * 2 + lax.axis_index("c")
        lane = lax.iota(jnp.int32, _L)
        neg_inf = jnp.full((_L,), -jnp.inf, jnp.float32)

        base = (2 * w) // 8 * 8
        pltpu.sync_copy(mo_hbm.at[pl.ds(base, 8)], mov)

        def start(t, slot):
            b_i, chunk = divmod(t, _NCHUNK)
            return pltpu.async_copy(
                ft_hbm.at[2 * w + b_i, pl.ds(chunk, 1)], bufs[slot], sems[slot]
            )

        cps = [start(t, t) for t in range(_NBUF)]

        def topk_row(r):
            rv = jnp.full((_L,), r, jnp.int32)

            def lane_chunk(g, l0):
                cv = l0 + _L * (_L * g + lane)
                v = plsc.load_gather(mov, [rv, jnp.minimum(cv, _C - 1)])
                return jnp.where(cv < _C, v, neg_inf), cv

            vmax = neg_inf
            vidx = jnp.zeros((_L,), jnp.int32)
            for c in range(_NCH):
                cv = _L * c + lane
                v = plsc.load_gather(mov, [rv, jnp.minimum(cv, _C - 1)])
                v = jnp.where(cv < _C, v, neg_inf)
                upd = v > vmax
                vmax = jnp.where(upd, v, vmax)
                vidx = jnp.where(upd, cv, vidx)

            def step(t, carry):
                vmax, vidx, ca, cb = carry
                m = lax.reduce_max(vmax, (0,))
                cstar = lax.reduce_min(
                    jnp.where(vmax == m, vidx, jnp.int32(2 * _C)), (0,))
                j = t - _DROP
                ca = jnp.where((j >= 0) & (lane == j), cstar, ca)
                cb = jnp.where(lane == j - _L, cstar, cb)
                lstar = cstar % _L
                plsc.store_scatter(mov, [rv, jnp.full((_L,), cstar, jnp.int32)],
                                   neg_inf, mask=lane == 0)
                lm = neg_inf
                li = jnp.zeros((_L,), jnp.int32)
                for g in range(4):
                    v, cv = lane_chunk(g, lstar)
                    upd = v > lm
                    lm = jnp.where(upd, v, lm)
                    li = jnp.where(upd, cv, li)
                ml = lax.reduce_max(lm, (0,))
                cl = lax.reduce_min(
                    jnp.where(lm == ml, li, jnp.int32(2 * _C)), (0,))
                vmax = jnp.where(lane == lstar, ml, vmax)
                vidx = jnp.where(lane == lstar, cl, vidx)
                return vmax, vidx, ca, cb

            zi = jnp.zeros((_L,), jnp.int32)
            _, _, ca, cb = lax.fori_loop(
                0, _TOPK, step, (vmax, vidx, zi, zi))
            return ca, cb

        r0 = 2 * w - base
        ca0, cb0 = topk_row(r0)
        ca1, cb1 = topk_row(r0 + 1)

        tail_on = lane < (_KEEP - _L)
        z = jnp.zeros((_L,), jnp.int32)

        def reduce_chunk(t, slot, total):
            b_i = t // _NCHUNK
            buf = bufs[slot]
            ca = jnp.where(b_i == 0, ca0, ca1)
            cb = jnp.where(b_i == 0, cb0, cb1)
            cbc = jnp.minimum(cb, _C - 1)

            def srow(s2, acc):
                s2v = jnp.broadcast_to(s2, (_L,)).astype(jnp.int32)
                ga = plsc.load_gather(buf, [z, s2v, ca])
                gb = plsc.load_gather(buf, [z, s2v, cbc])
                return acc + ga + jnp.where(tail_on, gb, jnp.float32(0.0))

            return lax.fori_loop(0, _S, srow, total)

        total = jnp.zeros((_L,), jnp.float32)
        for t in range(_NITER):
            cps[t % _NBUF].wait()
            if t + _NBUF < _NITER:
                cps[t % _NBUF] = start(t + _NBUF, t % _NBUF)
            total = reduce_chunk(t, t % _NBUF, total)

        accv[...] = total
        pltpu.sync_copy(accv, out_hbm.at[w])

    return k(ft, main_out)


def kernel(p, main_out, features):
    pt = jnp.transpose(p, (0, 2, 3, 1)).reshape(_ROWS, _C)
    ft = jnp.transpose(features, (0, 2, 3, 1))
    partials = _sc_gather_sum(ft, main_out)
    sum_p, gsum_tc = _psum_feat(main_out, pt, ft)
    denom = jnp.float32(_B * _S * _S)
    gsum = gsum_tc[0, 0] + jnp.sum(partials)
    loss = sum_p[0, 0] / denom + gsum / (denom * _KEEP)
    return jnp.abs(loss)


# speedup vs baseline: 1.0362x; 1.0163x over previous
import functools

import jax
import jax.numpy as jnp
from jax import lax
from jax.experimental import pallas as pl
from jax.experimental.pallas import tpu as pltpu
from jax.experimental.pallas import tpu_sc as plsc

_TOPK = 25
_DROP = 3
_KEEP = _TOPK - _DROP

_B = 64
_C = 1000
_S = 24
_S_SC = 12
_S_TC = _S - _S_SC

_ROWS = _B * _S * _S

_GRID = 8
_BB = _B // _GRID
_PR = _ROWS // _GRID

_NW = 32
_L = 16
_NCH = 63

_NBUF = 4
_NCHUNK = _S_SC
_NITER = 2 * _NCHUNK


def _fused_body(mo_ref, p_ref, f_ref, sum_ref, gsum_ref, pacc_ref, gacc_ref,
                mask_ref):
    i = pl.program_id(0)

    @pl.when(i == 0)
    def _():
        pacc_ref[...] = jnp.zeros_like(pacc_ref)
        gacc_ref[...] = jnp.zeros_like(gacc_ref)
        vals = mo_ref[...]
        col = lax.broadcasted_iota(jnp.int32, (_B, _C), 1)
        mask = jnp.zeros((_B, _C), jnp.float32)
        for t in range(_TOPK):
            m = jnp.max(vals, axis=1, keepdims=True)
            cand = jnp.where(vals == m, col, jnp.int32(_C))
            amin = jnp.min(cand, axis=1, keepdims=True)
            hit = col == amin
            if t >= _DROP:
                mask = mask + jnp.where(hit, jnp.float32(1.0), jnp.float32(0.0))
            vals = jnp.where(hit, -jnp.inf, vals)
        mask_ref[...] = mask.reshape(_GRID, _BB, _C)

    x = p_ref[...].reshape(_PR // 8, 8, _C)
    pacc_ref[...] += jnp.sum(x, axis=0)

    f = f_ref[...].reshape(_BB, _S_TC * _S, _C)
    m = mask_ref[i].reshape(_BB, 1, _C)
    fm = (f * m).reshape(_BB * _S_TC * _S // 8, 8, _C)
    gacc_ref[...] += jnp.sum(fm, axis=0)

    @pl.when(i == _GRID - 1)
    def _():
        sum_ref[0, 0] = jnp.sum(pacc_ref[...])
        gsum_ref[0, 0] = jnp.sum(gacc_ref[...])


def _psum_feat(main_out, pt, ft):
    return pl.pallas_call(
        _fused_body,
        grid=(_GRID,),
        in_specs=[
            pl.BlockSpec((_B, _C), lambda i: (0, 0)),
            pl.BlockSpec((_PR, _C), lambda i: (i, 0)),
            pl.BlockSpec((_BB, _S_TC, _S, _C), lambda i: (i, _S_SC // _S_TC, 0, 0)),
        ],
        out_specs=[
            pl.BlockSpec(memory_space=pltpu.SMEM),
            pl.BlockSpec(memory_space=pltpu.SMEM),
        ],
        out_shape=[
            jax.ShapeDtypeStruct((1, 1), jnp.float32),
            jax.ShapeDtypeStruct((1, 1), jnp.float32),
        ],
        scratch_shapes=[
            pltpu.VMEM((8, _C), jnp.float32),
            pltpu.VMEM((8, _C), jnp.float32),
            pltpu.VMEM((_GRID, _BB, _C), jnp.float32),
        ],
    )(main_out, pt, ft)


def _sc_gather_sum(ft, main_out):
    mesh = plsc.VectorSubcoreMesh(core_axis_name="c", subcore_axis_name="s")

    @functools.partial(
        pl.kernel,
        mesh=mesh,
        compiler_params=pltpu.CompilerParams(needs_layout_passes=False),
        out_type=jax.ShapeDtypeStruct((_NW, _L), jnp.float32),
        scratch_types=(
            [pltpu.VMEM((8, _C), jnp.float32)]
            + [pltpu.VMEM((1, _S, _C), jnp.float32)] * _NBUF
            + [pltpu.VMEM((_L,), jnp.float32)]
            + [pltpu.SemaphoreType.DMA] * _NBUF
        ),
    )
    def k(ft_hbm, mo_hbm, out_hbm, mov, *rest):
        bufs = rest[:_NBUF]
        accv = rest[_NBUF]
        sems = rest[_NBUF + 1 :]
        w = lax.axis_index("s") * 2 + lax.axis_index("c")
        lane = lax.iota(jnp.int32, _L)
        neg_inf = jnp.full((_L,), -jnp.inf, jnp.float32)

        base = (2 * w) // 8 * 8
        pltpu.sync_copy(mo_hbm.at[pl.ds(base, 8)], mov)

        def start(t, slot):
            b_i, chunk = divmod(t, _NCHUNK)
            return pltpu.async_copy(
                ft_hbm.at[2 * w + b_i, pl.ds(chunk, 1)], bufs[slot], sems[slot]
            )

        cps = [start(t, t) for t in range(_NBUF)]

        def topk_row(r):
            rv = jnp.full((_L,), r, jnp.int32)

            def lane_chunk(g, l0):
                cv = l0 + _L * (_L * g + lane)
                v = plsc.load_gather(mov, [rv, jnp.minimum(cv, _C - 1)])
                return jnp.where(cv < _C, v, neg_inf), cv

            vmax = neg_inf
            vidx = jnp.zeros((_L,), jnp.int32)
            for c in range(_NCH):
                cv = _L * c + lane
                v = plsc.load_gather(mov, [rv, jnp.minimum(cv, _C - 1)])
                v = jnp.where(cv < _C, v, neg_inf)
                upd = v > vmax
                vmax = jnp.where(upd, v, vmax)
                vidx = jnp.where(upd, cv, vidx)

            def step(t, carry):
                vmax, vidx, ca, cb = carry
                m = lax.reduce_max(vmax, (0,))
                cstar = lax.reduce_min(
                    jnp.where(vmax == m, vidx, jnp.int32(2 * _C)), (0,))
                j = t - _DROP
                ca = jnp.where((j >= 0) & (lane == j), cstar, ca)
                cb = jnp.where(lane == j - _L, cstar, cb)
                lstar = cstar % _L
                plsc.store_scatter(mov, [rv, jnp.full((_L,), cstar, jnp.int32)],
                                   neg_inf, mask=lane == 0)
                lm = neg_inf
                li = jnp.zeros((_L,), jnp.int32)
                for g in range(4):
                    v, cv = lane_chunk(g, lstar)
                    upd = v > lm
                    lm = jnp.where(upd, v, lm)
                    li = jnp.where(upd, cv, li)
                ml = lax.reduce_max(lm, (0,))
                cl = lax.reduce_min(
                    jnp.where(lm == ml, li, jnp.int32(2 * _C)), (0,))
                vmax = jnp.where(lane == lstar, ml, vmax)
                vidx = jnp.where(lane == lstar, cl, vidx)
                return vmax, vidx, ca, cb

            zi = jnp.zeros((_L,), jnp.int32)
            _, _, ca, cb = lax.fori_loop(
                0, _TOPK, step, (vmax, vidx, zi, zi))
            return ca, cb

        r0 = 2 * w - base
        ca0, cb0 = topk_row(r0)
        ca1, cb1 = topk_row(r0 + 1)

        tail_on = lane < (_KEEP - _L)
        z = jnp.zeros((_L,), jnp.int32)

        def reduce_chunk(t, slot, total):
            b_i = t // _NCHUNK
            buf = bufs[slot]
            ca = jnp.where(b_i == 0, ca0, ca1)
            cb = jnp.where(b_i == 0, cb0, cb1)
            cbc = jnp.minimum(cb, _C - 1)

            def srow(s2, acc):
                s2v = jnp.broadcast_to(s2, (_L,)).astype(jnp.int32)
                ga = plsc.load_gather(buf, [z, s2v, ca])
                gb = plsc.load_gather(buf, [z, s2v, cbc])
                return acc + ga + jnp.where(tail_on, gb, jnp.float32(0.0))

            return lax.fori_loop(0, _S, srow, total)

        total = jnp.zeros((_L,), jnp.float32)
        for t in range(_NITER):
            cps[t % _NBUF].wait()
            if t + _NBUF < _NITER:
                cps[t % _NBUF] = start(t + _NBUF, t % _NBUF)
            total = reduce_chunk(t, t % _NBUF, total)

        accv[...] = total
        pltpu.sync_copy(accv, out_hbm.at[w])

    return k(ft, main_out)


def kernel(p, main_out, features):
    pt = jnp.transpose(p, (0, 2, 3, 1)).reshape(_ROWS, _C)
    ft = jnp.transpose(features, (0, 2, 3, 1))
    partials = _sc_gather_sum(ft, main_out)
    sum_p, gsum_tc = _psum_feat(main_out, pt, ft)
    denom = jnp.float32(_B * _S * _S)
    gsum = gsum_tc[0, 0] + jnp.sum(partials)
    loss = sum_p[0, 0] / denom + gsum / (denom * _KEEP)
    return jnp.abs(loss)
